# baseline (speedup 1.0000x reference)
import functools

import jax
import jax.numpy as jnp
from jax import lax
from jax.experimental import pallas as pl
from jax.experimental.pallas import tpu as pltpu
from jax.experimental.pallas import tpu_sc as plsc

INNER = 32
D_MODEL = 128


def _tc_pack_transpose(table_t, vocab, q):
    BC = 16384
    grid = q // BC
    last_blk = pl.cdiv(vocab, BC) - 1

    def body(x0, x1, x2, x3, o_ref):
        o_ref[...] = jnp.concatenate(
            (x0[...], x1[...], x2[...], x3[...]), axis=0).T

    def imap(k):
        return lambda i: (0, jnp.minimum(i + k * grid, last_blk))

    return pl.pallas_call(
        body,
        grid=(grid,),
        in_specs=[pl.BlockSpec((INNER, BC), imap(k)) for k in range(4)],
        out_specs=pl.BlockSpec((BC, 128), lambda i: (i, 0)),
        out_shape=jax.ShapeDtypeStruct((q, 128), jnp.float32),
    )(table_t, table_t, table_t, table_t)


def _sc_gather(table_lin, flat_ids, n_tok):
    info = plsc.get_sparse_core_info()
    nw = info.num_cores * info.num_subcores
    nc = info.num_cores
    rows_per_w = n_tok // nw
    chunk = 1280
    n_chunks = rows_per_w // chunk

    mesh = plsc.VectorSubcoreMesh(core_axis_name="c", subcore_axis_name="s")

    @functools.partial(
        pl.kernel,
        out_type=jax.ShapeDtypeStruct((n_tok, 128), jnp.float32),
        mesh=mesh,
        compiler_params=pltpu.CompilerParams(use_tc_tiling_on_sc=False),
        scratch_types=[
            pltpu.VMEM((rows_per_w,), jnp.int32),
            pltpu.VMEM((chunk, INNER), jnp.float32),
            pltpu.VMEM((chunk, INNER), jnp.float32),
            pltpu.SemaphoreType.DMA,
            pltpu.SemaphoreType.DMA,
        ],
    )
    def gather_kernel(table_hbm, idx_hbm, out_hbm, idx_v, rows0, rows1, sem0, sem1):
        wid = lax.axis_index("s") * nc + lax.axis_index("c")
        base = wid * rows_per_w
        pltpu.sync_copy(idx_hbm.at[pl.ds(base, rows_per_w)], idx_v)
        rows = (rows0, rows1)
        sems = (sem0, sem1)
        cps = [None, None]
        cps[0] = pltpu.async_copy(
            table_hbm.at[idx_v.at[pl.ds(0, chunk)]], rows[0], sems[0])
        for c in range(n_chunks):
            if c + 1 < n_chunks:
                cps[(c + 1) % 2] = pltpu.async_copy(
                    table_hbm.at[idx_v.at[pl.ds((c + 1) * chunk, chunk)]],
                    rows[(c + 1) % 2], sems[(c + 1) % 2])
            cps[c % 2].wait()
            pltpu.sync_copy(
                rows[c % 2],
                out_hbm.at[pl.ds(base + c * chunk, chunk), pl.ds(0, INNER)])

    return gather_kernel(table_lin, flat_ids)


def _tc_project(gathered_pad, w_t, n_tok):
    TM = 12800
    grid = n_tok // TM

    def mm(x_ref, w_ref, o_ref):
        o_ref[...] = jnp.dot(x_ref[:, 0:INNER], w_ref[...],
                             preferred_element_type=jnp.float32)

    return pl.pallas_call(
        mm,
        grid=(grid,),
        in_specs=[
            pl.BlockSpec((TM, 128), lambda i: (i, 0)),
            pl.BlockSpec((INNER, D_MODEL), lambda i: (0, 0)),
        ],
        out_specs=pl.BlockSpec((TM, D_MODEL), lambda i: (i, 0)),
        out_shape=jax.ShapeDtypeStruct((n_tok, D_MODEL), jnp.float32),
    )(gathered_pad, w_t)


def kernel(token_ids, emb_table, W):
    b, l = token_ids.shape
    n_tok = b * l
    vocab = emb_table.shape[0]
    flat_ids = token_ids.T.reshape(n_tok)
    q = 16 * 16384
    lin_ids = 4 * (flat_ids % q) + flat_ids // q
    table_packed = _tc_pack_transpose(emb_table.T, vocab, q)
    table_lin = table_packed.reshape(4 * q, INNER)
    gathered = _sc_gather(table_lin, lin_ids, n_tok)
    y = _tc_project(gathered, W.T, n_tok)
    return y.reshape(l, b, D_MODEL).transpose(1, 0, 2)


# speedup vs baseline: 1.1299x; 1.1299x over previous
import functools

import jax
import jax.numpy as jnp
from jax import lax
from jax.experimental import pallas as pl
from jax.experimental.pallas import tpu as pltpu
from jax.experimental.pallas import tpu_sc as plsc

INNER = 32
D_MODEL = 128


def _tc_pack_transpose(table_t, vocab, q):
    BC = 16384
    grid = q // BC
    last_blk = pl.cdiv(vocab, BC) - 1

    def body(x0, x1, x2, x3, o_ref):
        o_ref[...] = jnp.concatenate(
            (x0[...], x1[...], x2[...], x3[...]), axis=0).T

    def imap(k):
        return lambda i: (0, jnp.minimum(i + k * grid, last_blk))

    return pl.pallas_call(
        body,
        grid=(grid,),
        in_specs=[pl.BlockSpec((INNER, BC), imap(k)) for k in range(4)],
        out_specs=pl.BlockSpec((BC, 128), lambda i: (i, 0)),
        out_shape=jax.ShapeDtypeStruct((q, 128), jnp.float32),
    )(table_t, table_t, table_t, table_t)


def _sc_gather(table_lin, flat_ids, n_tok):
    info = plsc.get_sparse_core_info()
    nw = info.num_cores * info.num_subcores
    nc = info.num_cores
    rows_per_w = n_tok // nw
    chunk = 1280
    n_chunks = rows_per_w // chunk

    mesh = plsc.VectorSubcoreMesh(core_axis_name="c", subcore_axis_name="s")

    @functools.partial(
        pl.kernel,
        out_type=jax.ShapeDtypeStruct((n_tok, INNER), jnp.float32),
        mesh=mesh,
        compiler_params=pltpu.CompilerParams(use_tc_tiling_on_sc=False),
        scratch_types=[
            pltpu.VMEM((rows_per_w,), jnp.int32),
            pltpu.VMEM((chunk, INNER), jnp.float32),
            pltpu.VMEM((chunk, INNER), jnp.float32),
            pltpu.SemaphoreType.DMA,
            pltpu.SemaphoreType.DMA,
        ],
    )
    def gather_kernel(table_hbm, idx_hbm, out_hbm, idx_v, rows0, rows1, sem0, sem1):
        wid = lax.axis_index("s") * nc + lax.axis_index("c")
        base = wid * rows_per_w
        pltpu.sync_copy(idx_hbm.at[pl.ds(base, rows_per_w)], idx_v)
        rows = (rows0, rows1)
        sems = (sem0, sem1)
        cps = [None, None]
        cps[0] = pltpu.async_copy(
            table_hbm.at[idx_v.at[pl.ds(0, chunk)]], rows[0], sems[0])
        for c in range(n_chunks):
            if c + 1 < n_chunks:
                cps[(c + 1) % 2] = pltpu.async_copy(
                    table_hbm.at[idx_v.at[pl.ds((c + 1) * chunk, chunk)]],
                    rows[(c + 1) % 2], sems[(c + 1) % 2])
            cps[c % 2].wait()
            pltpu.sync_copy(rows[c % 2],
                            out_hbm.at[pl.ds(base + c * chunk, chunk)])

    return gather_kernel(table_lin, flat_ids)


def _tc_project(gathered_grouped, w_t, n_tok):
    TM = 12800
    TG = TM // 4
    grid = n_tok // TM

    def mm(x_ref, w_ref, o_ref):
        x = x_ref[...]
        for k in range(4):
            yk = jnp.dot(x[:, 32 * k:32 * k + 32], w_ref[...],
                         preferred_element_type=jnp.float32)
            o_ref[pl.Slice(k, TG, 4), :] = yk

    return pl.pallas_call(
        mm,
        grid=(grid,),
        in_specs=[
            pl.BlockSpec((TG, 128), lambda i: (i, 0)),
            pl.BlockSpec((INNER, D_MODEL), lambda i: (0, 0)),
        ],
        out_specs=pl.BlockSpec((TM, D_MODEL), lambda i: (i, 0)),
        out_shape=jax.ShapeDtypeStruct((n_tok, D_MODEL), jnp.float32),
    )(gathered_grouped, w_t)


def kernel(token_ids, emb_table, W):
    b, l = token_ids.shape
    n_tok = b * l
    vocab = emb_table.shape[0]
    flat_ids = token_ids.T.reshape(n_tok)
    q = 16 * 16384
    lin_ids = 4 * (flat_ids % q) + flat_ids // q
    table_packed = _tc_pack_transpose(emb_table.T, vocab, q)
    table_lin = table_packed.reshape(4 * q, INNER)
    gathered = _sc_gather(table_lin, lin_ids, n_tok)
    y = _tc_project(gathered.reshape(n_tok // 4, 128), W.T, n_tok)
    return y.reshape(l, b, D_MODEL).transpose(1, 0, 2)
